# baseline (speedup 1.0000x reference)
import functools

import jax
import jax.numpy as jnp
from jax import lax
from jax.experimental import pallas as pl
from jax.experimental.pallas import tpu as pltpu
from jax.experimental.pallas import tpu_sc as plsc

N = 10000
E = 320000
D = 128
H = 8
C = 16
G = 64

NP = 10240
NP8 = NP // 8
NE = E + N
NW = 32
PER_W = 10368
EP = NW * PER_W
EPAD = EP - NE
K = 96
NCHUNK = PER_W // K
BR = 80
NG = K // 16
RT = NP // 16
ZR = 160

f32 = jnp.float32
i32 = jnp.int32


def _sc_edge_body(xp_hbm, ts_hbm, td_hbm, src_hbm, dst_hbm,
                  acc_out, s_out,
                  acc_sp, s_sp,
                  sidx, didx,
                  rows_v0, tsr_v0, tdr_v0, w_v0,
                  rows_v1, tsr_v1, tdr_v1, w_v1,
                  semg0, semg1, sems0, sems1):
    cid = lax.axis_index("c")
    tid = lax.axis_index("s")
    wid = tid * 2 + cid
    tb = tid * RT

    bufs = ((rows_v0, tsr_v0, tdr_v0, w_v0, semg0, sems0),
            (rows_v1, tsr_v1, tdr_v1, w_v1, semg1, sems1))

    def zb_row(i, _):
        for j in range(8):
            rows_v0[i, pl.ds(16 * j, 16)] = jnp.zeros((16,), f32)
        return 0
    lax.fori_loop(0, K, zb_row, 0)

    def zw_row(i, _):
        w_v0[i, :] = jnp.zeros((16,), f32)
        return 0
    lax.fori_loop(0, K, zw_row, 0)

    for r in range(RT // BR):
        pltpu.sync_copy(rows_v0.at[pl.ds(0, BR)],
                        acc_sp.at[pl.ds(tb + r * BR, BR)])
        pltpu.sync_copy(w_v0.at[pl.ds(0, BR)],
                        s_sp.at[pl.ds(tb + r * BR, BR)])
    plsc.subcore_barrier()

    def fetch(ci, b):
        rows_v, tsr_v, tdr_v, w_v, semg, sems = bufs[b]
        p = lax.shift_right_logical(ci, 2) & 1
        j = ci & 3

        @pl.when(j == 0)
        def _():
            row0 = wid * NCHUNK + ci
            pltpu.sync_copy(src_hbm.at[pl.ds(row0, 4)], sidx.at[p])
            pltpu.sync_copy(dst_hbm.at[pl.ds(row0, 4)], didx.at[p])
        pltpu.async_copy(xp_hbm.at[sidx.at[p, j]], rows_v, semg)
        pltpu.async_copy(ts_hbm.at[sidx.at[p, j]], tsr_v, semg)
        pltpu.async_copy(td_hbm.at[didx.at[p, j]], tdr_v, semg)

    def wait_scatter(b):
        rows_v, tsr_v, tdr_v, w_v, semg, sems = bufs[b]
        pltpu.make_async_copy(rows_v, acc_sp.at[didx.at[0, 0]], sems).wait()
        pltpu.make_async_copy(w_v, s_sp.at[didx.at[0, 0]], sems).wait()

    def fetch_w(ci, b):
        wait_scatter(b)
        fetch(ci, b)

    def drain(ci, b):
        rows_v, tsr_v, tdr_v, w_v, semg, sems = bufs[b]
        p = lax.shift_right_logical(ci, 2) & 1
        j = ci & 3
        pltpu.make_async_copy(xp_hbm.at[sidx.at[p, j]], rows_v, semg).wait()
        pltpu.make_async_copy(ts_hbm.at[sidx.at[p, j]], tsr_v, semg).wait()
        pltpu.make_async_copy(td_hbm.at[didx.at[p, j]], tdr_v, semg).wait()

    def consume(ci, b):
        rows_v, tsr_v, tdr_v, w_v, semg, sems = bufs[b]
        drain(ci, b)

        @plsc.parallel_loop(0, K, 1, unroll=8)
        def _edge(e):
            a = tsr_v[e, :] + tdr_v[e, :]
            al = jnp.where(a > 0, a, 0.2 * a)
            w = jnp.exp(al)
            w_v[e, :] = w
            for h in range(H):
                seg = rows_v[e, pl.ds(16 * h, 16)]
                rows_v[e, pl.ds(16 * h, 16)] = seg * w[h]

        p = lax.shift_right_logical(ci, 2) & 1
        j = ci & 3
        pltpu.async_copy(rows_v, acc_sp.at[didx.at[p, j]], sems, add=True)
        pltpu.async_copy(w_v, s_sp.at[didx.at[p, j]], sems, add=True)

    fetch(0, 0)
    fetch(1, 1)

    def step(t, _):
        consume(2 * t, 0)

        @pl.when(2 * t + 2 < NCHUNK)
        def _():
            fetch_w(2 * t + 2, 0)
        consume(2 * t + 1, 1)

        @pl.when(2 * t + 3 < NCHUNK)
        def _():
            fetch_w(2 * t + 3, 1)
        return 0
    lax.fori_loop(0, NCHUNK // 2, step, 0)

    wait_scatter(0)
    wait_scatter(1)
    plsc.subcore_barrier()

    for r in range(RT // BR):
        pltpu.sync_copy(acc_sp.at[pl.ds(tb + r * BR, BR)],
                        rows_v0.at[pl.ds(0, BR)])
        pltpu.sync_copy(rows_v0.at[pl.ds(0, BR)],
                        acc_out.at[cid, pl.ds(tb + r * BR, BR)])
        pltpu.sync_copy(s_sp.at[pl.ds(tb + r * BR, BR)],
                        w_v0.at[pl.ds(0, BR)])
        pltpu.sync_copy(w_v0.at[pl.ds(0, BR)],
                        s_out.at[cid, pl.ds(tb + r * BR, BR)])


@functools.lru_cache(maxsize=1)
def _sc_edge_kernel():
    return functools.partial(
        pl.kernel,
        mesh=plsc.VectorSubcoreMesh(core_axis_name="c", subcore_axis_name="s"),
        compiler_params=pltpu.CompilerParams(use_tc_tiling_on_sc=False),
        out_type=[
            jax.ShapeDtypeStruct((2, NP, D), f32),
            jax.ShapeDtypeStruct((2, NP, 16), f32),
        ],
        scratch_types=[
            pltpu.VMEM_SHARED((NP, D), f32),
            pltpu.VMEM_SHARED((NP, 16), f32),
            pltpu.VMEM((2, 4, K), i32),
            pltpu.VMEM((2, 4, K), i32),
            pltpu.VMEM((K, D), f32),
            pltpu.VMEM((K, 16), f32),
            pltpu.VMEM((K, 16), f32),
            pltpu.VMEM((K, 16), f32),
            pltpu.VMEM((K, D), f32),
            pltpu.VMEM((K, 16), f32),
            pltpu.VMEM((K, 16), f32),
            pltpu.VMEM((K, 16), f32),
            pltpu.SemaphoreType.DMA,
            pltpu.SemaphoreType.DMA,
            pltpu.SemaphoreType.DMA,
            pltpu.SemaphoreType.DMA,
        ],
    )(_sc_edge_body)


def _sc_edge(xp, ts, td, src, dst):
    return _sc_edge_kernel()(xp, ts, td,
                             src.reshape(EP // K, K), dst.reshape(EP // K, K))


def _tc_pre_body(x_ref, w_ref, as2_ref, ad2_ref, xp_ref, ts_ref, td_ref):
    xp = jnp.dot(x_ref[...], w_ref[...], preferred_element_type=f32)
    xp_ref[...] = xp
    ts_ref[...] = jnp.dot(xp, as2_ref[...], preferred_element_type=f32)
    td_ref[...] = jnp.dot(xp, ad2_ref[...], preferred_element_type=f32)


def _tc_pre(xpad, w, as2, ad2):
    return pl.pallas_call(
        _tc_pre_body,
        out_shape=[
            jax.ShapeDtypeStruct((NP, D), f32),
            jax.ShapeDtypeStruct((NP, 16), f32),
            jax.ShapeDtypeStruct((NP, 16), f32),
        ],
    )(xpad, w, as2, ad2)


def _combine_bn_elu(acc2, s2, prev, b, g, be, e16):
    acc = acc2[0] + acc2[1]
    s16 = s2[0] + s2[1]
    s_exp = jnp.dot(s16, e16, preferred_element_type=f32)
    res = acc / (s_exp + 1e-16) + b + prev
    real = res[:N]
    mu = jnp.mean(real, axis=0)
    dv = real - mu
    var = jnp.mean(dv * dv, axis=0)
    hn = (res - mu) * lax.rsqrt(var + 1e-5) * g + be
    hfull = jnp.where(hn > 0, hn, jnp.exp(jnp.minimum(hn, 0.0)) - 1.0)
    rowid = lax.broadcasted_iota(i32, (NP, 1), 0)
    return jnp.where(rowid < N, hfull, 0.0)


def _tc_post_body(acc2_ref, s2_ref, prev_ref, b_ref, g_ref, be_ref,
                  wn_ref, as2n_ref, ad2n_ref, e16_ref,
                  h_ref, xp_ref, ts_ref, td_ref):
    h = _combine_bn_elu(acc2_ref[...], s2_ref[...], prev_ref[...],
                        b_ref[...], g_ref[...], be_ref[...], e16_ref[...])
    h_ref[...] = h
    xp = jnp.dot(h, wn_ref[...], preferred_element_type=f32)
    xp_ref[...] = xp
    ts_ref[...] = jnp.dot(xp, as2n_ref[...], preferred_element_type=f32)
    td_ref[...] = jnp.dot(xp, ad2n_ref[...], preferred_element_type=f32)


def _tc_post(acc2, s2, prev, b, g, be, wn, as2n, ad2n, e16):
    return pl.pallas_call(
        _tc_post_body,
        out_shape=[
            jax.ShapeDtypeStruct((NP, D), f32),
            jax.ShapeDtypeStruct((NP, D), f32),
            jax.ShapeDtypeStruct((NP, 16), f32),
            jax.ShapeDtypeStruct((NP, 16), f32),
        ],
    )(acc2, s2, prev, b, g, be, wn, as2n, ad2n, e16)


def _tc_final_body(acc2_ref, s2_ref, prev_ref, b_ref, g_ref, be_ref,
                   e16_ref, batch_ref, wr_ref, br_ref, out_ref):
    h = _combine_bn_elu(acc2_ref[...], s2_ref[...], prev_ref[...],
                        b_ref[...], g_ref[...], be_ref[...], e16_ref[...])
    hr = h[:N]
    gid = lax.broadcasted_iota(i32, (G, N), 0)
    oh = (gid == batch_ref[...]).astype(f32)
    sums = jnp.dot(oh, hr, preferred_element_type=f32)
    cnt = jnp.sum(oh, axis=1, keepdims=True)
    pooled = sums / jnp.maximum(cnt, 1.0)
    out_ref[...] = jnp.dot(pooled, wr_ref[...],
                           preferred_element_type=f32) + br_ref[...]


def _tc_final(acc2, s2, prev, b, g, be, e16, batch2d, wr, br):
    return pl.pallas_call(
        _tc_final_body,
        out_shape=jax.ShapeDtypeStruct((G, 2), f32),
    )(acc2, s2, prev, b, g, be, e16, batch2d, wr, br)


def _attn_mat2(a):
    m = (a[:, :, None] * jnp.eye(H, dtype=f32)[:, None, :]).reshape(D, H)
    return jnp.concatenate([m, m], axis=1)


def kernel(x, W0, as0, ad0, b0, g0, be0, W1, as1, ad1, b1, g1, be1,
           W2, as2, ad2, b2, g2, be2, Wr, br, edge_index, batch):
    xpad = jnp.pad(x, ((0, NP - N), (0, 0)))
    loop = jnp.arange(N, dtype=edge_index.dtype)
    padv = N + (jnp.arange(EPAD, dtype=jnp.int32) % (NP - N))
    src = jnp.concatenate([edge_index[0], loop, padv]).astype(i32)
    dst = jnp.concatenate([edge_index[1], loop, padv]).astype(i32)

    e8 = (jnp.eye(H, dtype=f32)[:, :, None]
          * jnp.ones((1, 1, C), f32)).reshape(H, D)
    e16 = jnp.concatenate([e8, jnp.zeros((H, D), f32)], axis=0)
    batch2d = batch.reshape(1, N).astype(i32)

    params = [
        (W0, _attn_mat2(as0), _attn_mat2(ad0),
         b0.reshape(1, D), g0.reshape(1, D), be0.reshape(1, D)),
        (W1, _attn_mat2(as1), _attn_mat2(ad1),
         b1.reshape(1, D), g1.reshape(1, D), be1.reshape(1, D)),
        (W2, _attn_mat2(as2), _attn_mat2(ad2),
         b2.reshape(1, D), g2.reshape(1, D), be2.reshape(1, D)),
    ]

    xp, ts, td = _tc_pre(xpad, params[0][0], params[0][1], params[0][2])
    acc2, s2 = _sc_edge(xp, ts, td, src, dst)
    prev0 = jnp.zeros((NP, D), f32)
    h0, xp, ts, td = _tc_post(acc2, s2, prev0, params[0][3], params[0][4],
                              params[0][5], params[1][0], params[1][1],
                              params[1][2], e16)
    acc2, s2 = _sc_edge(xp, ts, td, src, dst)
    h1, xp, ts, td = _tc_post(acc2, s2, xpad, params[1][3], params[1][4],
                              params[1][5], params[2][0], params[2][1],
                              params[2][2], e16)
    acc2, s2 = _sc_edge(xp, ts, td, src, dst)
    return _tc_final(acc2, s2, h0, params[2][3], params[2][4], params[2][5],
                     e16, batch2d, Wr, br)


# speedup vs baseline: 1.4664x; 1.4664x over previous
import functools

import jax
import jax.numpy as jnp
from jax import lax
from jax.experimental import pallas as pl
from jax.experimental.pallas import tpu as pltpu
from jax.experimental.pallas import tpu_sc as plsc

N = 10000
E = 320000
D = 128
H = 8
C = 16
G = 64

NP = 10240
NP8 = NP // 8
NE = E + N
NW = 32
PER_W = 10368
EP = NW * PER_W
EPAD = EP - NE
K = 96
NCHUNK = PER_W // K
BR = 80
NG = K // 16
RT = NP // 16
ZR = 160

f32 = jnp.float32
i32 = jnp.int32


def _sc_edge_body(xp_hbm, ts_hbm, td_hbm, src_hbm, dst_hbm,
                  acc_out, s_out,
                  acc_sp, s_sp,
                  sidx, didx,
                  rows_v0, tsr_v0, tdr_v0, w_v0,
                  rows_v1, tsr_v1, tdr_v1, w_v1,
                  semg0, semg1, sems0, sems1):
    cid = lax.axis_index("c")
    tid = lax.axis_index("s")
    wid = tid * 2 + cid
    tb = tid * RT

    bufs = ((rows_v0, tsr_v0, tdr_v0, w_v0, semg0, sems0),
            (rows_v1, tsr_v1, tdr_v1, w_v1, semg1, sems1))

    def zb_row(i, _):
        for j in range(8):
            rows_v0[i, pl.ds(16 * j, 16)] = jnp.zeros((16,), f32)
        return 0
    lax.fori_loop(0, K, zb_row, 0)

    def zw_row(i, _):
        w_v0[i, :] = jnp.zeros((16,), f32)
        return 0
    lax.fori_loop(0, K, zw_row, 0)

    for r in range(RT // BR):
        pltpu.sync_copy(rows_v0.at[pl.ds(0, BR)],
                        acc_sp.at[pl.ds(tb + r * BR, BR)])
        pltpu.sync_copy(w_v0.at[pl.ds(0, BR)],
                        s_sp.at[pl.ds(tb + r * BR, BR)])
    plsc.subcore_barrier()

    def fetch(ci, b):
        rows_v, tsr_v, tdr_v, w_v, semg, sems = bufs[b]
        p = lax.shift_right_logical(ci, 2) & 1
        j = ci & 3

        @pl.when(j == 0)
        def _():
            row0 = wid * NCHUNK + ci
            pltpu.sync_copy(src_hbm.at[pl.ds(row0, 4)], sidx.at[p])
            pltpu.sync_copy(dst_hbm.at[pl.ds(row0, 4)], didx.at[p])
        pltpu.async_copy(xp_hbm.at[sidx.at[p, j]], rows_v, semg)
        pltpu.async_copy(ts_hbm.at[sidx.at[p, j]], tsr_v, semg)
        pltpu.async_copy(td_hbm.at[didx.at[p, j]], tdr_v, semg)

    def wait_scatter(b):
        rows_v, tsr_v, tdr_v, w_v, semg, sems = bufs[b]
        pltpu.make_async_copy(rows_v, acc_sp.at[didx.at[0, 0]], sems).wait()
        pltpu.make_async_copy(w_v, s_sp.at[didx.at[0, 0]], sems).wait()

    def fetch_w(ci, b):
        wait_scatter(b)
        fetch(ci, b)

    def drain(ci, b):
        rows_v, tsr_v, tdr_v, w_v, semg, sems = bufs[b]
        p = lax.shift_right_logical(ci, 2) & 1
        j = ci & 3
        pltpu.make_async_copy(xp_hbm.at[sidx.at[p, j]], rows_v, semg).wait()
        pltpu.make_async_copy(ts_hbm.at[sidx.at[p, j]], tsr_v, semg).wait()
        pltpu.make_async_copy(td_hbm.at[didx.at[p, j]], tdr_v, semg).wait()

    def consume(ci, b):
        rows_v, tsr_v, tdr_v, w_v, semg, sems = bufs[b]
        drain(ci, b)

        @plsc.parallel_loop(0, K, 1, unroll=4)
        def _edge(e):
            a = tsr_v[e, :] + tdr_v[e, :]
            al = jnp.where(a > 0, a, 0.2 * a)
            w = jnp.exp(al)
            w_v[e, :] = w
            for h in range(H):
                seg = rows_v[e, pl.ds(16 * h, 16)]
                rows_v[e, pl.ds(16 * h, 16)] = seg * w[h]

        p = lax.shift_right_logical(ci, 2) & 1
        j = ci & 3
        pltpu.async_copy(rows_v, acc_sp.at[didx.at[p, j]], sems, add=True)
        pltpu.async_copy(w_v, s_sp.at[didx.at[p, j]], sems, add=True)

    fetch(0, 0)
    fetch(1, 1)

    def step(t, _):
        consume(2 * t, 0)

        @pl.when(2 * t + 2 < NCHUNK)
        def _():
            fetch_w(2 * t + 2, 0)
        consume(2 * t + 1, 1)

        @pl.when(2 * t + 3 < NCHUNK)
        def _():
            fetch_w(2 * t + 3, 1)
        return 0
    lax.fori_loop(0, NCHUNK // 2, step, 0)

    wait_scatter(0)
    wait_scatter(1)
    plsc.subcore_barrier()

    for r in range(RT // BR):
        pltpu.sync_copy(acc_sp.at[pl.ds(tb + r * BR, BR)],
                        rows_v0.at[pl.ds(0, BR)])
        pltpu.sync_copy(rows_v0.at[pl.ds(0, BR)],
                        acc_out.at[cid, pl.ds(tb + r * BR, BR)])
        pltpu.sync_copy(s_sp.at[pl.ds(tb + r * BR, BR)],
                        w_v0.at[pl.ds(0, BR)])
        pltpu.sync_copy(w_v0.at[pl.ds(0, BR)],
                        s_out.at[cid, pl.ds(tb + r * BR, BR)])


@functools.lru_cache(maxsize=1)
def _sc_edge_kernel():
    return functools.partial(
        pl.kernel,
        mesh=plsc.VectorSubcoreMesh(core_axis_name="c", subcore_axis_name="s"),
        compiler_params=pltpu.CompilerParams(use_tc_tiling_on_sc=False),
        out_type=[
            jax.ShapeDtypeStruct((2, NP, D), f32),
            jax.ShapeDtypeStruct((2, NP, 16), f32),
        ],
        scratch_types=[
            pltpu.VMEM_SHARED((NP, D), f32),
            pltpu.VMEM_SHARED((NP, 16), f32),
            pltpu.VMEM((2, 4, K), i32),
            pltpu.VMEM((2, 4, K), i32),
            pltpu.VMEM((K, D), f32),
            pltpu.VMEM((K, 16), f32),
            pltpu.VMEM((K, 16), f32),
            pltpu.VMEM((K, 16), f32),
            pltpu.VMEM((K, D), f32),
            pltpu.VMEM((K, 16), f32),
            pltpu.VMEM((K, 16), f32),
            pltpu.VMEM((K, 16), f32),
            pltpu.SemaphoreType.DMA,
            pltpu.SemaphoreType.DMA,
            pltpu.SemaphoreType.DMA,
            pltpu.SemaphoreType.DMA,
        ],
    )(_sc_edge_body)


def _sc_edge(xp, ts, td, src, dst):
    return _sc_edge_kernel()(xp, ts, td,
                             src.reshape(EP // K, K), dst.reshape(EP // K, K))


def _tc_pre_body(x_ref, w_ref, as2_ref, ad2_ref, xp_ref, ts_ref, td_ref):
    xp = jnp.dot(x_ref[...], w_ref[...], preferred_element_type=f32)
    xp_ref[...] = xp
    ts_ref[...] = jnp.dot(xp, as2_ref[...], preferred_element_type=f32)
    td_ref[...] = jnp.dot(xp, ad2_ref[...], preferred_element_type=f32)


def _tc_pre(xpad, w, as2, ad2):
    return pl.pallas_call(
        _tc_pre_body,
        out_shape=[
            jax.ShapeDtypeStruct((NP, D), f32),
            jax.ShapeDtypeStruct((NP, 16), f32),
            jax.ShapeDtypeStruct((NP, 16), f32),
        ],
    )(xpad, w, as2, ad2)


def _combine_bn_elu(acc2, s2, prev, b, g, be, e16):
    acc = acc2[0] + acc2[1]
    s16 = s2[0] + s2[1]
    s_exp = jnp.dot(s16, e16, preferred_element_type=f32)
    res = acc / (s_exp + 1e-16) + b + prev
    real = res[:N]
    mu = jnp.mean(real, axis=0)
    dv = real - mu
    var = jnp.mean(dv * dv, axis=0)
    hn = (res - mu) * lax.rsqrt(var + 1e-5) * g + be
    hfull = jnp.where(hn > 0, hn, jnp.exp(jnp.minimum(hn, 0.0)) - 1.0)
    rowid = lax.broadcasted_iota(i32, (NP, 1), 0)
    return jnp.where(rowid < N, hfull, 0.0)


def _tc_post_body(acc2_ref, s2_ref, prev_ref, b_ref, g_ref, be_ref,
                  wn_ref, as2n_ref, ad2n_ref, e16_ref,
                  h_ref, xp_ref, ts_ref, td_ref):
    h = _combine_bn_elu(acc2_ref[...], s2_ref[...], prev_ref[...],
                        b_ref[...], g_ref[...], be_ref[...], e16_ref[...])
    h_ref[...] = h
    xp = jnp.dot(h, wn_ref[...], preferred_element_type=f32)
    xp_ref[...] = xp
    ts_ref[...] = jnp.dot(xp, as2n_ref[...], preferred_element_type=f32)
    td_ref[...] = jnp.dot(xp, ad2n_ref[...], preferred_element_type=f32)


def _tc_post(acc2, s2, prev, b, g, be, wn, as2n, ad2n, e16):
    return pl.pallas_call(
        _tc_post_body,
        out_shape=[
            jax.ShapeDtypeStruct((NP, D), f32),
            jax.ShapeDtypeStruct((NP, D), f32),
            jax.ShapeDtypeStruct((NP, 16), f32),
            jax.ShapeDtypeStruct((NP, 16), f32),
        ],
    )(acc2, s2, prev, b, g, be, wn, as2n, ad2n, e16)


def _tc_final_body(acc2_ref, s2_ref, prev_ref, b_ref, g_ref, be_ref,
                   e16_ref, batch_ref, wr_ref, br_ref, out_ref):
    h = _combine_bn_elu(acc2_ref[...], s2_ref[...], prev_ref[...],
                        b_ref[...], g_ref[...], be_ref[...], e16_ref[...])
    hr = h[:N]
    gid = lax.broadcasted_iota(i32, (G, N), 0)
    oh = (gid == batch_ref[...]).astype(f32)
    sums = jnp.dot(oh, hr, preferred_element_type=f32)
    cnt = jnp.sum(oh, axis=1, keepdims=True)
    pooled = sums / jnp.maximum(cnt, 1.0)
    out_ref[...] = jnp.dot(pooled, wr_ref[...],
                           preferred_element_type=f32) + br_ref[...]


def _tc_final(acc2, s2, prev, b, g, be, e16, batch2d, wr, br):
    return pl.pallas_call(
        _tc_final_body,
        out_shape=jax.ShapeDtypeStruct((G, 2), f32),
    )(acc2, s2, prev, b, g, be, e16, batch2d, wr, br)


def _attn_mat2(a):
    m = (a[:, :, None] * jnp.eye(H, dtype=f32)[:, None, :]).reshape(D, H)
    return jnp.concatenate([m, m], axis=1)


def kernel(x, W0, as0, ad0, b0, g0, be0, W1, as1, ad1, b1, g1, be1,
           W2, as2, ad2, b2, g2, be2, Wr, br, edge_index, batch):
    xpad = jnp.pad(x, ((0, NP - N), (0, 0)))
    loop = jnp.arange(N, dtype=edge_index.dtype)
    padv = N + (jnp.arange(EPAD, dtype=jnp.int32) % (NP - N))
    src = jnp.concatenate([edge_index[0], loop, padv]).astype(i32)
    dst = jnp.concatenate([edge_index[1], loop, padv]).astype(i32)

    e8 = (jnp.eye(H, dtype=f32)[:, :, None]
          * jnp.ones((1, 1, C), f32)).reshape(H, D)
    e16 = jnp.concatenate([e8, jnp.zeros((H, D), f32)], axis=0)
    batch2d = batch.reshape(1, N).astype(i32)

    params = [
        (W0, _attn_mat2(as0), _attn_mat2(ad0),
         b0.reshape(1, D), g0.reshape(1, D), be0.reshape(1, D)),
        (W1, _attn_mat2(as1), _attn_mat2(ad1),
         b1.reshape(1, D), g1.reshape(1, D), be1.reshape(1, D)),
        (W2, _attn_mat2(as2), _attn_mat2(ad2),
         b2.reshape(1, D), g2.reshape(1, D), be2.reshape(1, D)),
    ]

    xp, ts, td = _tc_pre(xpad, params[0][0], params[0][1], params[0][2])
    acc2, s2 = _sc_edge(xp, ts, td, src, dst)
    prev0 = jnp.zeros((NP, D), f32)
    h0, xp, ts, td = _tc_post(acc2, s2, prev0, params[0][3], params[0][4],
                              params[0][5], params[1][0], params[1][1],
                              params[1][2], e16)
    acc2, s2 = _sc_edge(xp, ts, td, src, dst)
    h1, xp, ts, td = _tc_post(acc2, s2, xpad, params[1][3], params[1][4],
                              params[1][5], params[2][0], params[2][1],
                              params[2][2], e16)
    acc2, s2 = _sc_edge(xp, ts, td, src, dst)
    return _tc_final(acc2, s2, h0, params[2][3], params[2][4], params[2][5],
                     e16, batch2d, Wr, br)
